# baseline (speedup 1.0000x reference)
import functools

import jax
import jax.numpy as jnp
from jax import lax
from jax.experimental import pallas as pl
from jax.experimental.pallas import tpu as pltpu
from jax.experimental.pallas import tpu_sc as plsc

BATCH = 16384
IND_DIM = 32
STY_DIM = 16
REG_DIM = 16
OUT_DIM = IND_DIM + STY_DIM + REG_DIM

NUM_CORES = 2
NUM_SUBCORES = 16
NUM_WORKERS = NUM_CORES * NUM_SUBCORES
B_PER_W = 512
HALF = B_PER_W // 2
CHUNK = 128
NCH = HALF // CHUNK
W_ROWS = BATCH // 2


def _emb_body(ind_hbm, sty_hbm, reg_hbm, w_ind, w_sty, w_reg, out_hbm,
              idx_i, idx_s, idx_r, rows_i, rows_s, rows_r, sem):
    wid = lax.axis_index("s") * NUM_CORES + lax.axis_index("c")
    wbase = wid * HALF

    pltpu.sync_copy(ind_hbm.at[wid], idx_i)
    pltpu.sync_copy(sty_hbm.at[wid], idx_s)
    pltpu.sync_copy(reg_hbm.at[wid], idx_r)

    copies = []
    for par in range(2):
        for j in range(NCH):
            sl = pl.ds(j * CHUNK, CHUNK)
            copies.append(pltpu.async_copy(
                w_ind.at[idx_i.at[par, j]], rows_i.at[par, sl], sem))
            copies.append(pltpu.async_copy(
                w_sty.at[idx_s.at[par, j]], rows_s.at[par, sl], sem))
            copies.append(pltpu.async_copy(
                w_reg.at[idx_r.at[par, j]], rows_r.at[par, sl], sem))
    for c in copies:
        c.wait()

    rows = pl.ds(wbase, HALF)
    for par in range(2):
        off = par * OUT_DIM
        pltpu.sync_copy(rows_i.at[par], out_hbm.at[rows, pl.ds(off, IND_DIM)])
        pltpu.sync_copy(rows_s.at[par],
                        out_hbm.at[rows, pl.ds(off + IND_DIM, STY_DIM)])
        pltpu.sync_copy(rows_r.at[par],
                        out_hbm.at[rows, pl.ds(off + IND_DIM + STY_DIM, REG_DIM)])


_launch = functools.partial(
    pl.kernel,
    out_type=jax.ShapeDtypeStruct((W_ROWS, 2 * OUT_DIM), jnp.float32),
    mesh=plsc.VectorSubcoreMesh(core_axis_name="c", subcore_axis_name="s"),
    compiler_params=pltpu.CompilerParams(use_tc_tiling_on_sc=False),
    scratch_types=[
        pltpu.VMEM((2, NCH, CHUNK), jnp.int32),
        pltpu.VMEM((2, NCH, CHUNK), jnp.int32),
        pltpu.VMEM((2, NCH, CHUNK), jnp.int32),
        pltpu.VMEM((2, HALF, IND_DIM), jnp.float32),
        pltpu.VMEM((2, HALF, STY_DIM), jnp.float32),
        pltpu.VMEM((2, HALF, REG_DIM), jnp.float32),
        pltpu.SemaphoreType.DMA,
    ],
)(_emb_body)


def _split_parity(idx):
    x = idx.astype(jnp.int32).reshape(NUM_WORKERS, HALF, 2)
    return x.transpose(0, 2, 1).reshape(NUM_WORKERS, 2, NCH, CHUNK)


@jax.jit
def kernel(industry_idx, style_idx, regime_idx, W_industry, W_style, W_regime):
    out_wide = _launch(_split_parity(industry_idx), _split_parity(style_idx),
                       _split_parity(regime_idx), W_industry, W_style, W_regime)
    return out_wide.reshape(BATCH, OUT_DIM)


# speedup vs baseline: 1.0632x; 1.0632x over previous
import functools

import jax
import jax.numpy as jnp
from jax import lax
from jax.experimental import pallas as pl
from jax.experimental.pallas import tpu as pltpu
from jax.experimental.pallas import tpu_sc as plsc

BATCH = 16384
IND_DIM = 32
STY_DIM = 16
REG_DIM = 16
OUT_DIM = IND_DIM + STY_DIM + REG_DIM

NUM_CORES = 2
NUM_SUBCORES = 16
NUM_WORKERS = NUM_CORES * NUM_SUBCORES
B_PER_W = BATCH // NUM_WORKERS
CHUNK = 128
NCHUNK = B_PER_W // CHUNK


def _emb_body(ind_hbm, sty_hbm, reg_hbm, w_ind, w_sty, w_reg, out_hbm,
              idx_i, idx_s, idx_r, rows_i, rows_s, rows_r, sem):
    wid = lax.axis_index("s") * NUM_CORES + lax.axis_index("c")
    base = wid * B_PER_W

    pltpu.sync_copy(ind_hbm.at[wid], idx_i)
    pltpu.sync_copy(sty_hbm.at[wid], idx_s)
    pltpu.sync_copy(reg_hbm.at[wid], idx_r)

    copies = []
    for j in range(NCHUNK):
        sl = pl.ds(j * CHUNK, CHUNK)
        copies.append(pltpu.async_copy(w_ind.at[idx_i.at[j]], rows_i.at[sl], sem))
        copies.append(pltpu.async_copy(w_sty.at[idx_s.at[j]], rows_s.at[sl], sem))
        copies.append(pltpu.async_copy(w_reg.at[idx_r.at[j]], rows_r.at[sl], sem))
    for c in copies:
        c.wait()

    rows = pl.ds(base, B_PER_W)
    pltpu.sync_copy(rows_i, out_hbm.at[rows, pl.ds(0, IND_DIM)])
    pltpu.sync_copy(rows_s, out_hbm.at[rows, pl.ds(IND_DIM, STY_DIM)])
    pltpu.sync_copy(rows_r, out_hbm.at[rows, pl.ds(IND_DIM + STY_DIM, REG_DIM)])


_launch = functools.partial(
    pl.kernel,
    out_type=jax.ShapeDtypeStruct((BATCH, OUT_DIM), jnp.float32),
    mesh=plsc.VectorSubcoreMesh(core_axis_name="c", subcore_axis_name="s"),
    compiler_params=pltpu.CompilerParams(use_tc_tiling_on_sc=False),
    scratch_types=[
        pltpu.VMEM((NCHUNK, CHUNK), jnp.int32),
        pltpu.VMEM((NCHUNK, CHUNK), jnp.int32),
        pltpu.VMEM((NCHUNK, CHUNK), jnp.int32),
        pltpu.VMEM((B_PER_W, IND_DIM), jnp.float32),
        pltpu.VMEM((B_PER_W, STY_DIM), jnp.float32),
        pltpu.VMEM((B_PER_W, REG_DIM), jnp.float32),
        pltpu.SemaphoreType.DMA,
    ],
)(_emb_body)


@jax.jit
def kernel(industry_idx, style_idx, regime_idx, W_industry, W_style, W_regime):
    shape3 = (NUM_WORKERS, NCHUNK, CHUNK)
    ind = industry_idx.astype(jnp.int32).reshape(shape3)
    sty = style_idx.astype(jnp.int32).reshape(shape3)
    reg = regime_idx.astype(jnp.int32).reshape(shape3)
    return _launch(ind, sty, reg, W_industry, W_style, W_regime)
